# Initial kernel scaffold; baseline (speedup 1.0000x reference)
import jax
import jax.numpy as jnp
from jax.experimental import pallas as pl


def kernel(inputs):
    raise NotImplementedError("write your pallas kernel here")


# speedup vs baseline: 10.8599x; 10.8599x over previous
import functools

import jax
import jax.numpy as jnp
from jax.experimental import pallas as pl
from jax.experimental.pallas import tpu as pltpu

_N = 32768
_K = 2048
_RANK = _N - 1 - _K

_SIGN = -2147483648
_MANT = 0x7FFFFFFF


def _select_mask_kernel(x_ref, o_ref):
    x = x_ref[...]
    bits = jax.lax.bitcast_convert_type(x, jnp.int32)
    y = jnp.where(bits >= 0, bits, bits ^ _MANT)

    rows = x.shape[0]

    def body(i, prefix):
        b = 31 - i
        cand = prefix | jnp.left_shift(jnp.int32(1), b)
        cand_s = cand ^ _SIGN
        cnt = jnp.sum((y < cand_s).astype(jnp.int32), axis=-1, keepdims=True)
        return jnp.where(cnt <= _RANK, cand, prefix)

    prefix = jnp.zeros((rows, 1), jnp.int32)
    prefix = jax.lax.fori_loop(0, 32, body, prefix)

    t_s = prefix ^ _SIGN
    t_bits = jnp.where(t_s >= 0, t_s, t_s ^ _MANT)
    thr = jax.lax.bitcast_convert_type(t_bits, jnp.float32)
    o_ref[...] = x * (x > thr).astype(x.dtype)


@jax.jit
def kernel(inputs):
    rows, n = inputs.shape
    block_rows = 8
    grid = rows // block_rows
    return pl.pallas_call(
        _select_mask_kernel,
        grid=(grid,),
        in_specs=[pl.BlockSpec((block_rows, n), lambda i: (i, 0))],
        out_specs=pl.BlockSpec((block_rows, n), lambda i: (i, 0)),
        out_shape=jax.ShapeDtypeStruct((rows, n), inputs.dtype),
    )(inputs)
